# Initial kernel scaffold; baseline (speedup 1.0000x reference)
import jax
import jax.numpy as jnp
from jax.experimental import pallas as pl


def kernel(x, batch):
    raise NotImplementedError("write your pallas kernel here")


# speedup vs baseline: 3.3041x; 3.3041x over previous
import functools

import jax
import jax.numpy as jnp
from jax import lax
from jax.experimental import pallas as pl
from jax.experimental.pallas import tpu as pltpu
from jax.experimental.pallas import tpu_sc as plsc

N = 100000
C = 128
G = 64
CHUNK = 80
NSLOTS = N // CHUNK
NSUB = 16
NCORE = 2
CH = C // NCORE
GPT = G // NSUB
SLOTS_PER_TILE = -(-NSLOTS // NSUB)


def _body(x_hbm, b_hbm, out_hbm, idx_v, x_v, cnt_v, iota_v, zero_v,
          sums4_v, cnt64_v, out_v, sums_sh, cnt_sh):
    cid = lax.axis_index("c")
    sid = lax.axis_index("s")
    c0 = cid * CH
    g0 = sid * GPT

    zeros16f = jnp.zeros((16,), jnp.float32)
    ones16f = jnp.ones((16,), jnp.float32)

    for g in range(GPT):
        for j in range(CH // 16):
            zero_v[g, pl.ds(j * 16, 16)] = zeros16f
    for j in range(G // 16):
        cnt_v[pl.ds(j * 16, 16)] = zeros16f
        iota_v[pl.ds(j * 16, 16)] = lax.iota(jnp.int32, 16) + (16 * j)
    pltpu.sync_copy(zero_v, sums_sh.at[pl.ds(g0, GPT)])

    @pl.when(sid == 0)
    def _():
        pltpu.sync_copy(zero_v.at[0], cnt_sh)

    plsc.subcore_barrier()

    def step(i, _):
        slot = i * NSUB + sid

        @pl.when(slot < NSLOTS)
        def _():
            off = slot * CHUNK
            pltpu.sync_copy(b_hbm.at[pl.ds(off, CHUNK)], idx_v)
            pltpu.sync_copy(x_hbm.at[pl.ds(off, CHUNK), pl.ds(c0, CH)], x_v)
            pltpu.sync_copy(x_v, sums_sh.at[idx_v], add=True)
            for j in range(CHUNK // 16):
                iv = idx_v[pl.ds(j * 16, 16)]
                plsc.addupdate_scatter(cnt_v, [iv], ones16f)
        return _

    lax.fori_loop(0, SLOTS_PER_TILE, step, None)

    pltpu.sync_copy(cnt_v, cnt_sh.at[iota_v], add=True)
    plsc.subcore_barrier()

    pltpu.sync_copy(sums_sh.at[pl.ds(g0, GPT)], sums4_v)
    pltpu.sync_copy(cnt_sh, cnt64_v)
    for g in range(GPT):
        cvec = plsc.load_gather(cnt64_v, [jnp.full((16,), g0 + g, jnp.int32)])
        for j in range(CH // 16):
            out_v[g, pl.ds(j * 16, 16)] = sums4_v[g, pl.ds(j * 16, 16)] / cvec
    pltpu.sync_copy(out_v, out_hbm.at[pl.ds(g0, GPT), pl.ds(c0, CH)])


@jax.jit
def _pooling(x, batch):
    mesh = plsc.VectorSubcoreMesh(core_axis_name="c", subcore_axis_name="s")
    f = pl.kernel(
        _body,
        out_type=jax.ShapeDtypeStruct((G, C), jnp.float32),
        mesh=mesh,
        compiler_params=pltpu.CompilerParams(use_tc_tiling_on_sc=False,
                                             needs_layout_passes=False),
        scratch_types=[
            pltpu.VMEM((CHUNK,), jnp.int32),
            pltpu.VMEM((CHUNK, CH), jnp.float32),
            pltpu.VMEM((G,), jnp.float32),
            pltpu.VMEM((G,), jnp.int32),
            pltpu.VMEM((GPT, CH), jnp.float32),
            pltpu.VMEM((GPT, CH), jnp.float32),
            pltpu.VMEM((G,), jnp.float32),
            pltpu.VMEM((GPT, CH), jnp.float32),
            pltpu.VMEM_SHARED((G, CH), jnp.float32),
            pltpu.VMEM_SHARED((G,), jnp.float32),
        ],
    )
    return f(x, batch)


def kernel(x, batch):
    return _pooling(x, batch.astype(jnp.int32))


# speedup vs baseline: 7.1772x; 2.1722x over previous
import jax
import jax.numpy as jnp
from jax import lax
from jax.experimental import pallas as pl
from jax.experimental.pallas import tpu as pltpu
from jax.experimental.pallas import tpu_sc as plsc

N = 100000
C = 128
G = 64
CHUNK = 80
NSLOTS = N // CHUNK
NSUB = 16
NCORE = 2
CH = C // NCORE
GPT = G // NSUB
NBUF = 3
STEPS = -(-(NSLOTS // NSUB + 1) // NBUF)


def _body(x_hbm, b_hbm, out_hbm, idx_v, x_v, cnt_v, iota_v, zero_v,
          sums4_v, cnt64_v, out_v, sums_sh, cnt_sh, dsem, ssem):
    cid = lax.axis_index("c")
    sid = lax.axis_index("s")
    c0 = cid * CH
    g0 = sid * GPT

    zeros16f = jnp.zeros((16,), jnp.float32)
    ones16f = jnp.ones((16,), jnp.float32)

    for g in range(GPT):
        for j in range(CH // 16):
            zero_v[g, pl.ds(j * 16, 16)] = zeros16f
    for j in range(G // 16):
        cnt_v[pl.ds(j * 16, 16)] = zeros16f
        iota_v[pl.ds(j * 16, 16)] = lax.iota(jnp.int32, 16) + (16 * j)
    pltpu.sync_copy(zero_v, sums_sh.at[pl.ds(g0, GPT)])

    @pl.when(sid == 0)
    def _():
        pltpu.sync_copy(zero_v.at[0], cnt_sh)

    plsc.subcore_barrier()

    def issue(j, b):
        slot = j * NSUB + sid

        @pl.when(slot < NSLOTS)
        def _():
            off = slot * CHUNK
            pltpu.async_copy(b_hbm.at[pl.ds(off, CHUNK)], idx_v[b], dsem[b])
            pltpu.async_copy(
                x_hbm.at[pl.ds(off, CHUNK), pl.ds(c0, CH)], x_v[b], dsem[b])

    def process(j, b):
        slot = j * NSUB + sid

        @pl.when(slot < NSLOTS)
        def _():
            off = slot * CHUNK
            pltpu.make_async_copy(
                b_hbm.at[pl.ds(off, CHUNK)], idx_v[b], dsem[b]).wait()
            pltpu.make_async_copy(
                x_hbm.at[pl.ds(off, CHUNK), pl.ds(c0, CH)], x_v[b],
                dsem[b]).wait()
            sc = pltpu.async_copy(
                x_v[b], sums_sh.at[idx_v[b]], ssem, add=True)
            for q in range(CHUNK // 16):
                iv = idx_v[b][pl.ds(q * 16, 16)]
                plsc.addupdate_scatter(cnt_v, [iv], ones16f)
            sc.wait()

    for b in range(NBUF):
        issue(b, b)

    def step(i, _):
        for b in range(NBUF):
            j = i * NBUF + b
            process(j, b)
            issue(j + NBUF, b)
        return _

    lax.fori_loop(0, STEPS, step, None)

    pltpu.sync_copy(cnt_v, cnt_sh.at[iota_v], add=True)
    plsc.subcore_barrier()

    pltpu.sync_copy(sums_sh.at[pl.ds(g0, GPT)], sums4_v)
    pltpu.sync_copy(cnt_sh, cnt64_v)
    for g in range(GPT):
        cvec = plsc.load_gather(cnt64_v, [jnp.full((16,), g0 + g, jnp.int32)])
        for j in range(CH // 16):
            out_v[g, pl.ds(j * 16, 16)] = sums4_v[g, pl.ds(j * 16, 16)] / cvec
    pltpu.sync_copy(out_v, out_hbm.at[pl.ds(g0, GPT), pl.ds(c0, CH)])


@jax.jit
def _pooling(x, batch):
    mesh = plsc.VectorSubcoreMesh(core_axis_name="c", subcore_axis_name="s")
    f = pl.kernel(
        _body,
        out_type=jax.ShapeDtypeStruct((G, C), jnp.float32),
        mesh=mesh,
        compiler_params=pltpu.CompilerParams(use_tc_tiling_on_sc=False,
                                             needs_layout_passes=False),
        scratch_types=[
            [pltpu.VMEM((CHUNK,), jnp.int32) for _ in range(NBUF)],
            [pltpu.VMEM((CHUNK, CH), jnp.float32) for _ in range(NBUF)],
            pltpu.VMEM((G,), jnp.float32),
            pltpu.VMEM((G,), jnp.int32),
            pltpu.VMEM((GPT, CH), jnp.float32),
            pltpu.VMEM((GPT, CH), jnp.float32),
            pltpu.VMEM((G,), jnp.float32),
            pltpu.VMEM((GPT, CH), jnp.float32),
            pltpu.VMEM_SHARED((G, CH), jnp.float32),
            pltpu.VMEM_SHARED((G,), jnp.float32),
            [pltpu.SemaphoreType.DMA for _ in range(NBUF)],
            pltpu.SemaphoreType.DMA,
        ],
    )
    return f(x, batch)


def kernel(x, batch):
    return _pooling(x, batch.astype(jnp.int32))


# speedup vs baseline: 8.5790x; 1.1953x over previous
import jax
import jax.numpy as jnp
from jax import lax
from jax.experimental import pallas as pl
from jax.experimental.pallas import tpu as pltpu
from jax.experimental.pallas import tpu_sc as plsc

N = 100000
C = 128
G = 64
ROW = 80
SUBC = 5
CHUNK = ROW * SUBC
NSLOTS = N // CHUNK
NSUB = 16
NCORE = 2
CH = C // NCORE
GPT = G // NSUB
NBUF = 3
STEPS = -(-(NSLOTS // NSUB + 1) // NBUF)


def _body(x_hbm, b_hbm, out_hbm, idx_v, x_v, cnt_v, iota_v, zero_v,
          sums4_v, cnt64_v, out_v, sums_sh, cnt_sh, dsem, ssem):
    cid = lax.axis_index("c")
    sid = lax.axis_index("s")
    c0 = cid * CH
    g0 = sid * GPT

    zeros16f = jnp.zeros((16,), jnp.float32)
    ones16f = jnp.ones((16,), jnp.float32)

    for g in range(GPT):
        for j in range(CH // 16):
            zero_v[g, pl.ds(j * 16, 16)] = zeros16f
    for j in range(G // 16):
        cnt_v[pl.ds(j * 16, 16)] = zeros16f
        iota_v[pl.ds(j * 16, 16)] = lax.iota(jnp.int32, 16) + (16 * j)
    pltpu.sync_copy(zero_v, sums_sh.at[pl.ds(g0, GPT)])

    @pl.when(sid == 0)
    def _():
        pltpu.sync_copy(zero_v.at[0], cnt_sh)

    plsc.subcore_barrier()

    def issue(j, b):
        slot = j * NSUB + sid

        @pl.when(slot < NSLOTS)
        def _():
            off = slot * CHUNK
            pltpu.async_copy(
                b_hbm.at[pl.ds(slot * SUBC, SUBC), :], idx_v[b], dsem[b])
            pltpu.async_copy(
                x_hbm.at[pl.ds(off, CHUNK), pl.ds(c0, CH)], x_v[b], dsem[b])

    def process(j, b):
        slot = j * NSUB + sid

        @pl.when(slot < NSLOTS)
        def _():
            off = slot * CHUNK
            pltpu.make_async_copy(
                b_hbm.at[pl.ds(slot * SUBC, SUBC), :], idx_v[b],
                dsem[b]).wait()
            pltpu.make_async_copy(
                x_hbm.at[pl.ds(off, CHUNK), pl.ds(c0, CH)], x_v[b],
                dsem[b]).wait()
            scs = [
                pltpu.async_copy(
                    x_v[b].at[pl.ds(k * ROW, ROW)],
                    sums_sh.at[idx_v[b].at[k]], ssem, add=True)
                for k in range(SUBC)
            ]
            for k in range(SUBC):
                for q in range(ROW // 16):
                    iv = idx_v[b][k, pl.ds(q * 16, 16)]
                    plsc.addupdate_scatter(cnt_v, [iv], ones16f)
            for sc in scs:
                sc.wait()

    for b in range(NBUF):
        issue(b, b)

    def step(i, _):
        for b in range(NBUF):
            j = i * NBUF + b
            process(j, b)
            issue(j + NBUF, b)
        return _

    lax.fori_loop(0, STEPS, step, None)

    pltpu.sync_copy(cnt_v, cnt_sh.at[iota_v], add=True)
    plsc.subcore_barrier()

    pltpu.sync_copy(sums_sh.at[pl.ds(g0, GPT)], sums4_v)
    pltpu.sync_copy(cnt_sh, cnt64_v)
    for g in range(GPT):
        cvec = plsc.load_gather(cnt64_v, [jnp.full((16,), g0 + g, jnp.int32)])
        for j in range(CH // 16):
            out_v[g, pl.ds(j * 16, 16)] = sums4_v[g, pl.ds(j * 16, 16)] / cvec
    pltpu.sync_copy(out_v, out_hbm.at[pl.ds(g0, GPT), pl.ds(c0, CH)])


@jax.jit
def _pooling(x, batch):
    mesh = plsc.VectorSubcoreMesh(core_axis_name="c", subcore_axis_name="s")
    f = pl.kernel(
        _body,
        out_type=jax.ShapeDtypeStruct((G, C), jnp.float32),
        mesh=mesh,
        compiler_params=pltpu.CompilerParams(use_tc_tiling_on_sc=False,
                                             needs_layout_passes=False),
        scratch_types=[
            [pltpu.VMEM((SUBC, ROW), jnp.int32) for _ in range(NBUF)],
            [pltpu.VMEM((CHUNK, CH), jnp.float32) for _ in range(NBUF)],
            pltpu.VMEM((G,), jnp.float32),
            pltpu.VMEM((G,), jnp.int32),
            pltpu.VMEM((GPT, CH), jnp.float32),
            pltpu.VMEM((GPT, CH), jnp.float32),
            pltpu.VMEM((G,), jnp.float32),
            pltpu.VMEM((GPT, CH), jnp.float32),
            pltpu.VMEM_SHARED((G, CH), jnp.float32),
            pltpu.VMEM_SHARED((G,), jnp.float32),
            [pltpu.SemaphoreType.DMA for _ in range(NBUF)],
            pltpu.SemaphoreType.DMA,
        ],
    )
    return f(x, batch)


def kernel(x, batch):
    return _pooling(x, batch.astype(jnp.int32).reshape(N // ROW, ROW))
